# Initial kernel scaffold; baseline (speedup 1.0000x reference)
import jax
import jax.numpy as jnp
from jax.experimental import pallas as pl


def kernel(x, edge_index, edge_attr, W):
    raise NotImplementedError("write your pallas kernel here")


# speedup vs baseline: 7.0506x; 7.0506x over previous
import functools

import jax
import jax.numpy as jnp
from jax import lax
from jax.experimental import pallas as pl
from jax.experimental.pallas import tpu as pltpu
from jax.experimental.pallas import tpu_sc as plsc

N = 10000
E = 320000
D = 128
NUM_LAYERS = 4
TRI = (0, 1, 3, 6)

NC = 2
NS = 16
NW = NC * NS

EPW = E // NW
CHUNK = 80
NCHUNKS = EPW // CHUNK

NACC = 10240
ROWS_PER_TILE = NACC // NS
TRASH = N + 8

BM = 2000


def _matmul_body(x_ref, w_ref, h_ref):
    h_ref[0] = jnp.dot(x_ref[0], w_ref[0], preferred_element_type=jnp.float32)


def _build_h(xs_stack, w_all, t):
    grid = (t + 1, N // BM)
    return pl.pallas_call(
        _matmul_body,
        grid=grid,
        in_specs=[
            pl.BlockSpec((1, BM, D), lambda j, i, t=t: (t - j, i, 0)),
            pl.BlockSpec((1, D, D), lambda j, i, t=t: (TRI[t] + j, 0, 0)),
        ],
        out_specs=pl.BlockSpec((1, BM, D), lambda j, i: (j, i, 0)),
        out_shape=jax.ShapeDtypeStruct((NUM_LAYERS, N, D), jnp.float32),
    )(xs_stack, w_all)


def _edge_body(t, h_ref, attr_ref, src_ref, dst_ref, p_ref,
               a_v, s_v, d_v, gi_v, si_v, rows_v, zrow_v, acc, gsem):
    c = lax.axis_index("c")
    s = lax.axis_index("s")
    wid = c * NS + s

    zero16 = jnp.zeros((16,), jnp.float32)
    def zrow(i, carry):
        for j in range(D // 16):
            zrow_v[i, pl.ds(j * 16, 16)] = zero16
        return carry
    lax.fori_loop(0, 16, zrow, 0)

    row0 = s * ROWS_PER_TILE
    def zacc(r, carry):
        pltpu.sync_copy(zrow_v, acc.at[pl.ds(row0 + r * 16, 16)])
        return carry
    lax.fori_loop(0, ROWS_PER_TILE // 16, zacc, 0)
    plsc.subcore_barrier()

    base0 = wid * EPW
    kmax = t + 1

    def chunk(ci, carry):
        base = pl.multiple_of(base0 + ci * CHUNK, 16)
        pltpu.sync_copy(attr_ref.at[pl.ds(base, CHUNK)], a_v)
        pltpu.sync_copy(src_ref.at[pl.ds(base, CHUNK)], s_v)
        pltpu.sync_copy(dst_ref.at[pl.ds(base, CHUNK)], d_v)
        for i in range(CHUNK // 16):
            sl = pl.ds(i * 16, 16)
            a = a_v[sl]
            gi_v[sl] = (a - 1) * N + s_v[sl]
            si_v[sl] = jnp.where(a <= kmax, d_v[sl], TRASH)
        pltpu.async_copy(h_ref.at[gi_v], rows_v, gsem).wait()
        pltpu.sync_copy(rows_v, acc.at[si_v], add=True)
        return carry

    lax.fori_loop(0, NCHUNKS, chunk, 0)
    plsc.subcore_barrier()

    pltpu.sync_copy(acc.at[pl.ds(row0, ROWS_PER_TILE)],
                    p_ref.at[c, pl.ds(row0, ROWS_PER_TILE)])


def _edge_pass(h2, attr, src, dst, t):
    mesh = plsc.VectorSubcoreMesh(core_axis_name="c", subcore_axis_name="s",
                                  num_cores=NC, num_subcores=NS)
    kern = pl.kernel(
        functools.partial(_edge_body, t),
        out_type=jax.ShapeDtypeStruct((NC, NACC, D), jnp.float32),
        mesh=mesh,
        scratch_types=[
            pltpu.VMEM((CHUNK,), jnp.int32),
            pltpu.VMEM((CHUNK,), jnp.int32),
            pltpu.VMEM((CHUNK,), jnp.int32),
            pltpu.VMEM((CHUNK,), jnp.int32),
            pltpu.VMEM((CHUNK,), jnp.int32),
            pltpu.VMEM((CHUNK, D), jnp.float32),
            pltpu.VMEM((16, D), jnp.float32),
            pltpu.VMEM_SHARED((NACC, D), jnp.float32),
            pltpu.SemaphoreType.DMA,
        ],
    )
    return kern(h2, attr, src, dst)


def _combine_body(xs_ref, p_ref, o_ref):
    o_ref[0] = xs_ref[0] + jnp.maximum(p_ref[0] + p_ref[1], 0.0)


def _combine_next(xs_stack, p, t):
    return pl.pallas_call(
        _combine_body,
        grid=(N // BM,),
        in_specs=[
            pl.BlockSpec((1, BM, D), lambda i, t=t: (t, i, 0)),
            pl.BlockSpec((NC, BM, D), lambda i: (0, i, 0)),
        ],
        out_specs=pl.BlockSpec((1, BM, D), lambda i, t=t: (t + 1, i, 0)),
        out_shape=jax.ShapeDtypeStruct((NUM_LAYERS, N, D), jnp.float32),
        input_output_aliases={0: 0},
    )(xs_stack, p)


def _combine_final_body(xs_ref, p_ref, o_ref):
    o_ref[...] = xs_ref[0] + jnp.maximum(p_ref[0] + p_ref[1], 0.0)


def _combine_final(xs_stack, p):
    return pl.pallas_call(
        _combine_final_body,
        grid=(N // BM,),
        in_specs=[
            pl.BlockSpec((1, BM, D), lambda i: (NUM_LAYERS - 1, i, 0)),
            pl.BlockSpec((NC, BM, D), lambda i: (0, i, 0)),
        ],
        out_specs=pl.BlockSpec((BM, D), lambda i: (i, 0)),
        out_shape=jax.ShapeDtypeStruct((N, D), jnp.float32),
    )(xs_stack, p)


def kernel(x, edge_index, edge_attr, W):
    src = edge_index[0]
    dst = edge_index[1]
    xs_stack = jnp.zeros((NUM_LAYERS, N, D), jnp.float32).at[0].set(x)
    out = None
    for t in range(NUM_LAYERS):
        h = _build_h(xs_stack, W, t)
        h2 = h.reshape(NUM_LAYERS * N, D)
        p = _edge_pass(h2, edge_attr, src, dst, t)
        if t < NUM_LAYERS - 1:
            xs_stack = _combine_next(xs_stack, p, t)
        else:
            out = _combine_final(xs_stack, p)
    return out


# speedup vs baseline: 13.3195x; 1.8891x over previous
import functools

import jax
import jax.numpy as jnp
from jax import lax
from jax.experimental import pallas as pl
from jax.experimental.pallas import tpu as pltpu
from jax.experimental.pallas import tpu_sc as plsc

N = 10000
E = 320000
D = 128
NUM_LAYERS = 4
TRI = (0, 1, 3, 6)

NC = 2
NS = 16
NW = NC * NS

EPW = E // NW
CHUNK = 80
NCHUNKS = EPW // CHUNK

NACC = 10240
ROWS_PER_TILE = NACC // NS
TRASH = N + 8

BM = 2000


def _matmul_body(x_ref, w_ref, h_ref):
    h_ref[0] = jnp.dot(x_ref[0], w_ref[0], preferred_element_type=jnp.float32)


def _build_h(xs_stack, w_all, t):
    grid = (t + 1, N // BM)
    return pl.pallas_call(
        _matmul_body,
        grid=grid,
        in_specs=[
            pl.BlockSpec((1, BM, D), lambda j, i, t=t: (t - j, i, 0)),
            pl.BlockSpec((1, D, D), lambda j, i, t=t: (TRI[t] + j, 0, 0)),
        ],
        out_specs=pl.BlockSpec((1, BM, D), lambda j, i: (j, i, 0)),
        out_shape=jax.ShapeDtypeStruct((NUM_LAYERS, N, D), jnp.float32),
    )(xs_stack, w_all)


EBLK = 2000
CPB = EBLK // CHUNK
NBLK = EPW // EBLK


def _edge_body(t, h_ref, attr_ref, src_ref, dst_ref, p_ref,
               a_v, s_v, d_v, gi2, si2, rows0, rows1, acc,
               lsem, gsem0, gsem1):
    c = lax.axis_index("c")
    s = lax.axis_index("s")
    wid = c * NS + s
    base0 = pl.multiple_of(wid * EPW, 16)
    kmax = t + 1

    zero16 = jnp.zeros((16,), jnp.float32)
    def zrow(i, carry):
        for j in range(D // 16):
            rows0[i, pl.ds(j * 16, 16)] = zero16
        return carry
    lax.fori_loop(0, CHUNK, zrow, 0)

    row0 = s * ROWS_PER_TILE
    def zacc(r, carry):
        pltpu.sync_copy(rows0, acc.at[pl.ds(row0 + r * CHUNK, CHUNK)])
        return carry
    lax.fori_loop(0, ROWS_PER_TILE // CHUNK, zacc, 0)
    plsc.subcore_barrier()

    def gather(ci, rows, sem):
        return pltpu.async_copy(h_ref.at[gi2.at[ci]], rows, sem)

    def scatter(ci, rows):
        pltpu.sync_copy(rows, acc.at[si2.at[ci]], add=True)

    for blk in range(NBLK):
        bbase = pl.multiple_of(base0 + blk * EBLK, 16)
        la = pltpu.async_copy(attr_ref.at[pl.ds(bbase, EBLK)], a_v, lsem)
        ls = pltpu.async_copy(src_ref.at[pl.ds(bbase, EBLK)], s_v, lsem)
        ld = pltpu.async_copy(dst_ref.at[pl.ds(bbase, EBLK)], d_v, lsem)
        la.wait()
        ls.wait()
        ld.wait()

        def idx(ci, carry):
            for m in range(CHUNK // 16):
                sl = pl.ds(ci * CHUNK + m * 16, 16)
                a = a_v[sl]
                gi2[ci, pl.ds(m * 16, 16)] = (a - 1) * N + s_v[sl]
                si2[ci, pl.ds(m * 16, 16)] = jnp.where(a <= kmax, d_v[sl],
                                                       TRASH)
            return carry
        lax.fori_loop(0, CPB, idx, 0)

        def pair(j, carry):
            c0 = 2 * j
            d0 = gather(c0, rows0, gsem0)
            d1 = gather(c0 + 1, rows1, gsem1)
            d0.wait()
            scatter(c0, rows0)
            d1.wait()
            scatter(c0 + 1, rows1)
            return carry
        lax.fori_loop(0, CPB // 2, pair, 0)
        if CPB % 2:
            gather(CPB - 1, rows0, gsem0).wait()
            scatter(CPB - 1, rows0)

    plsc.subcore_barrier()
    pltpu.sync_copy(acc.at[pl.ds(row0, ROWS_PER_TILE)],
                    p_ref.at[c, pl.ds(row0, ROWS_PER_TILE)])


def _edge_pass(h2, attr, src, dst, t):
    mesh = plsc.VectorSubcoreMesh(core_axis_name="c", subcore_axis_name="s",
                                  num_cores=NC, num_subcores=NS)
    kern = pl.kernel(
        functools.partial(_edge_body, t),
        out_type=jax.ShapeDtypeStruct((NC, NACC, D), jnp.float32),
        mesh=mesh,
        scratch_types=[
            pltpu.VMEM((EBLK,), jnp.int32),
            pltpu.VMEM((EBLK,), jnp.int32),
            pltpu.VMEM((EBLK,), jnp.int32),
            pltpu.VMEM((CPB, CHUNK), jnp.int32),
            pltpu.VMEM((CPB, CHUNK), jnp.int32),
            pltpu.VMEM((CHUNK, D), jnp.float32),
            pltpu.VMEM((CHUNK, D), jnp.float32),
            pltpu.VMEM_SHARED((NACC, D), jnp.float32),
            pltpu.SemaphoreType.DMA,
            pltpu.SemaphoreType.DMA,
            pltpu.SemaphoreType.DMA,
        ],
    )
    return kern(h2, attr, src, dst)


def _combine_body(xs_ref, p_ref, o_ref):
    o_ref[0] = xs_ref[0] + jnp.maximum(p_ref[0] + p_ref[1], 0.0)


def _combine_next(xs_stack, p, t):
    return pl.pallas_call(
        _combine_body,
        grid=(N // BM,),
        in_specs=[
            pl.BlockSpec((1, BM, D), lambda i, t=t: (t, i, 0)),
            pl.BlockSpec((NC, BM, D), lambda i: (0, i, 0)),
        ],
        out_specs=pl.BlockSpec((1, BM, D), lambda i, t=t: (t + 1, i, 0)),
        out_shape=jax.ShapeDtypeStruct((NUM_LAYERS, N, D), jnp.float32),
        input_output_aliases={0: 0},
    )(xs_stack, p)


def _combine_final_body(xs_ref, p_ref, o_ref):
    o_ref[...] = xs_ref[0] + jnp.maximum(p_ref[0] + p_ref[1], 0.0)


def _combine_final(xs_stack, p):
    return pl.pallas_call(
        _combine_final_body,
        grid=(N // BM,),
        in_specs=[
            pl.BlockSpec((1, BM, D), lambda i: (NUM_LAYERS - 1, i, 0)),
            pl.BlockSpec((NC, BM, D), lambda i: (0, i, 0)),
        ],
        out_specs=pl.BlockSpec((BM, D), lambda i: (i, 0)),
        out_shape=jax.ShapeDtypeStruct((N, D), jnp.float32),
    )(xs_stack, p)


def kernel(x, edge_index, edge_attr, W):
    src = edge_index[0]
    dst = edge_index[1]
    xs_stack = jnp.zeros((NUM_LAYERS, N, D), jnp.float32).at[0].set(x)
    out = None
    for t in range(NUM_LAYERS):
        h = _build_h(xs_stack, W, t)
        h2 = h.reshape(NUM_LAYERS * N, D)
        p = _edge_pass(h2, edge_attr, src, dst, t)
        if t < NUM_LAYERS - 1:
            xs_stack = _combine_next(xs_stack, p, t)
        else:
            out = _combine_final(xs_stack, p)
    return out


# speedup vs baseline: 15.5339x; 1.1662x over previous
import functools

import jax
import jax.numpy as jnp
from jax import lax
from jax.experimental import pallas as pl
from jax.experimental.pallas import tpu as pltpu
from jax.experimental.pallas import tpu_sc as plsc

N = 10000
E = 320000
D = 128
NUM_LAYERS = 4
TRI = (0, 1, 3, 6)

NC = 2
NS = 16
NW = NC * NS

EPW = E // NW
CHUNK = 80
NCHUNKS = EPW // CHUNK

NACC = 10240
ROWS_PER_TILE = NACC // NS
TRASH = N + 8

BM = 2000


def _matmul_body(x_ref, w_ref, h_ref):
    h_ref[0] = jnp.dot(x_ref[0], w_ref[0], preferred_element_type=jnp.float32)


def _build_h(xs_stack, w_all, t):
    grid = (t + 1, N // BM)
    return pl.pallas_call(
        _matmul_body,
        grid=grid,
        in_specs=[
            pl.BlockSpec((1, BM, D), lambda j, i, t=t: (t - j, i, 0)),
            pl.BlockSpec((1, D, D), lambda j, i, t=t: (TRI[t] + j, 0, 0)),
        ],
        out_specs=pl.BlockSpec((1, BM, D), lambda j, i: (j, i, 0)),
        out_shape=jax.ShapeDtypeStruct((NUM_LAYERS, N, D), jnp.float32),
    )(xs_stack, w_all)


EBLK = 2000
CPB = EBLK // CHUNK
NBLK = EPW // EBLK


def _edge_body(t, h_ref, attr_ref, src_ref, dst_ref, p_ref,
               a_v, s_v, d_v, gi2, si2, rows0, rows1, acc,
               lsem, gsem0, gsem1):
    c = lax.axis_index("c")
    s = lax.axis_index("s")
    wid = c * NS + s
    base0 = pl.multiple_of(wid * EPW, 16)
    kmax = t + 1

    zero16 = jnp.zeros((16,), jnp.float32)
    def zrow(i, carry):
        for j in range(D // 16):
            rows0[i, pl.ds(j * 16, 16)] = zero16
        return carry
    lax.fori_loop(0, CHUNK, zrow, 0)

    row0 = s * ROWS_PER_TILE
    def zacc(r, carry):
        pltpu.sync_copy(rows0, acc.at[pl.ds(row0 + r * CHUNK, CHUNK)])
        return carry
    lax.fori_loop(0, ROWS_PER_TILE // CHUNK, zacc, 0)
    plsc.subcore_barrier()

    def gather(ci, rows, sem):
        return pltpu.async_copy(h_ref.at[gi2.at[ci]], rows, sem)

    def wait_gather(rows, sem):
        pltpu.make_async_copy(h_ref.at[gi2.at[0]], rows, sem).wait()

    def scatter(ci, rows):
        pltpu.sync_copy(rows, acc.at[si2.at[ci]], add=True)

    for blk in range(NBLK):
        bbase = pl.multiple_of(base0 + blk * EBLK, 16)
        la = pltpu.async_copy(attr_ref.at[pl.ds(bbase, EBLK)], a_v, lsem)
        ls = pltpu.async_copy(src_ref.at[pl.ds(bbase, EBLK)], s_v, lsem)
        ld = pltpu.async_copy(dst_ref.at[pl.ds(bbase, EBLK)], d_v, lsem)
        la.wait()
        ls.wait()
        ld.wait()

        def idx(ci, carry):
            for m in range(CHUNK // 16):
                sl = pl.ds(ci * CHUNK + m * 16, 16)
                a = a_v[sl]
                gi2[ci, pl.ds(m * 16, 16)] = (a - 1) * N + s_v[sl]
                si2[ci, pl.ds(m * 16, 16)] = jnp.where(a <= kmax, d_v[sl],
                                                       TRASH)
            return carry
        lax.fori_loop(0, CPB, idx, 0)

        gather(0, rows0, gsem0)

        def pair(j, carry):
            c0 = 2 * j
            gather(c0 + 1, rows1, gsem1)
            wait_gather(rows0, gsem0)
            scatter(c0, rows0)
            gather(c0 + 2, rows0, gsem0)
            wait_gather(rows1, gsem1)
            scatter(c0 + 1, rows1)
            return carry
        lax.fori_loop(0, CPB // 2, pair, 0)
        wait_gather(rows0, gsem0)
        scatter(CPB - 1, rows0)

    plsc.subcore_barrier()
    pltpu.sync_copy(acc.at[pl.ds(row0, ROWS_PER_TILE)],
                    p_ref.at[c, pl.ds(row0, ROWS_PER_TILE)])


def _edge_pass(h2, attr, src, dst, t):
    mesh = plsc.VectorSubcoreMesh(core_axis_name="c", subcore_axis_name="s",
                                  num_cores=NC, num_subcores=NS)
    kern = pl.kernel(
        functools.partial(_edge_body, t),
        out_type=jax.ShapeDtypeStruct((NC, NACC, D), jnp.float32),
        mesh=mesh,
        scratch_types=[
            pltpu.VMEM((EBLK,), jnp.int32),
            pltpu.VMEM((EBLK,), jnp.int32),
            pltpu.VMEM((EBLK,), jnp.int32),
            pltpu.VMEM((CPB, CHUNK), jnp.int32),
            pltpu.VMEM((CPB, CHUNK), jnp.int32),
            pltpu.VMEM((CHUNK, D), jnp.float32),
            pltpu.VMEM((CHUNK, D), jnp.float32),
            pltpu.VMEM_SHARED((NACC, D), jnp.float32),
            pltpu.SemaphoreType.DMA,
            pltpu.SemaphoreType.DMA,
            pltpu.SemaphoreType.DMA,
        ],
    )
    return kern(h2, attr, src, dst)


def _combine_body(xs_ref, p_ref, o_ref):
    o_ref[0] = xs_ref[0] + jnp.maximum(p_ref[0] + p_ref[1], 0.0)


def _combine_next(xs_stack, p, t):
    return pl.pallas_call(
        _combine_body,
        grid=(N // BM,),
        in_specs=[
            pl.BlockSpec((1, BM, D), lambda i, t=t: (t, i, 0)),
            pl.BlockSpec((NC, BM, D), lambda i: (0, i, 0)),
        ],
        out_specs=pl.BlockSpec((1, BM, D), lambda i, t=t: (t + 1, i, 0)),
        out_shape=jax.ShapeDtypeStruct((NUM_LAYERS, N, D), jnp.float32),
        input_output_aliases={0: 0},
    )(xs_stack, p)


def _combine_final_body(xs_ref, p_ref, o_ref):
    o_ref[...] = xs_ref[0] + jnp.maximum(p_ref[0] + p_ref[1], 0.0)


def _combine_final(xs_stack, p):
    return pl.pallas_call(
        _combine_final_body,
        grid=(N // BM,),
        in_specs=[
            pl.BlockSpec((1, BM, D), lambda i: (NUM_LAYERS - 1, i, 0)),
            pl.BlockSpec((NC, BM, D), lambda i: (0, i, 0)),
        ],
        out_specs=pl.BlockSpec((BM, D), lambda i: (i, 0)),
        out_shape=jax.ShapeDtypeStruct((N, D), jnp.float32),
    )(xs_stack, p)


def kernel(x, edge_index, edge_attr, W):
    src = edge_index[0]
    dst = edge_index[1]
    xs_stack = jnp.zeros((NUM_LAYERS, N, D), jnp.float32).at[0].set(x)
    out = None
    for t in range(NUM_LAYERS):
        h = _build_h(xs_stack, W, t)
        h2 = h.reshape(NUM_LAYERS * N, D)
        p = _edge_pass(h2, edge_attr, src, dst, t)
        if t < NUM_LAYERS - 1:
            xs_stack = _combine_next(xs_stack, p, t)
        else:
            out = _combine_final(xs_stack, p)
    return out
